# baseline (speedup 1.0000x reference)
import functools

import jax
import jax.numpy as jnp
from jax import lax
from jax.experimental import pallas as pl
from jax.experimental.pallas import tpu as pltpu
from jax.experimental.pallas import tpu_sc as plsc

N_NODES = 10000
N_EDGES = 320000
D_FEAT = 128
D_EDGE = 16
D_OUT = 2 * D_FEAT + D_EDGE

NUM_CORES = 2
NUM_SUBCORES = 16
NW = NUM_CORES * NUM_SUBCORES
B_PER_W = N_EDGES // NW
CHUNK = 80
N_CHUNKS = B_PER_W // CHUNK
RING = 5
N_ROUNDS = N_CHUNKS // RING
C_BIN = 200

_mesh = plsc.VectorSubcoreMesh(core_axis_name="c", subcore_axis_name="s")


@functools.partial(
    pl.kernel,
    mesh=_mesh,
    out_type=jax.ShapeDtypeStruct((N_EDGES, D_OUT), jnp.float32),
    scratch_types=(
        [pltpu.VMEM((CHUNK,), jnp.int32) for _ in range(2 * RING)]
        + [
            pltpu.VMEM((2 * CHUNK,), jnp.int32),
            pltpu.VMEM((RING, 2 * CHUNK, D_FEAT), jnp.float32),
            pltpu.VMEM((C_BIN, D_EDGE), jnp.float32),
            pltpu.SemaphoreType.DMA((RING,)),
            pltpu.SemaphoreType.DMA((RING,)),
            pltpu.SemaphoreType.DMA((RING,)),
            pltpu.SemaphoreType.DMA,
        ]
    ),
)
def _join_sc(unary, binary, index1, index2, out, *refs):
    i1s = refs[0:RING]
    i2s = refs[RING:2 * RING]
    dummy_i, g_v, bin_v, isem, gsem, wsem, bsem = refs[2 * RING:]

    wid = lax.axis_index("s") * NUM_CORES + lax.axis_index("c")
    w0 = wid * B_PER_W

    def start_idx(i, s):
        base = w0 + i * CHUNK
        pltpu.async_copy(index1.at[pl.ds(base, CHUNK)], i1s[s], isem.at[s])
        pltpu.async_copy(index2.at[pl.ds(base, CHUNK)], i2s[s], isem.at[s])

    def wait_idx(s):
        pltpu.make_async_copy(index1.at[pl.ds(w0, 2 * CHUNK)], dummy_i,
                              isem.at[s]).wait()

    def start_gathers(i, b):
        pltpu.async_copy(unary.at[i1s[b]],
                         g_v.at[b, pl.ds(0, CHUNK)], gsem.at[b])
        pltpu.async_copy(unary.at[i2s[b]],
                         g_v.at[b, pl.ds(CHUNK, CHUNK)], gsem.at[b])

    def drain_gathers(b):
        pltpu.make_async_copy(unary.at[i1s[b]],
                              g_v.at[b, pl.ds(0, CHUNK)], gsem.at[b]).wait()
        pltpu.make_async_copy(unary.at[i2s[b]],
                              g_v.at[b, pl.ds(CHUNK, CHUNK)], gsem.at[b]).wait()

    def start_writes(i, b):
        base = w0 + i * CHUNK
        pltpu.async_copy(g_v.at[b, pl.ds(0, CHUNK)],
                         out.at[pl.ds(base, CHUNK), pl.ds(0, D_FEAT)],
                         wsem.at[b])
        pltpu.async_copy(g_v.at[b, pl.ds(CHUNK, CHUNK)],
                         out.at[pl.ds(base, CHUNK), pl.ds(D_FEAT, D_FEAT)],
                         wsem.at[b])

    def drain_writes(b):
        pltpu.make_async_copy(g_v.at[b],
                              out.at[pl.ds(w0, 2 * CHUNK), pl.ds(0, D_FEAT)],
                              wsem.at[b]).wait()

    def bin_load(k):
        pltpu.async_copy(binary.at[pl.ds(w0 + k * C_BIN, C_BIN)], bin_v, bsem)

    def bin_drain_load():
        pltpu.make_async_copy(binary.at[pl.ds(w0, C_BIN)], bin_v, bsem).wait()

    def bin_write(k):
        pltpu.async_copy(
            bin_v,
            out.at[pl.ds(w0 + k * C_BIN, C_BIN), pl.ds(2 * D_FEAT, D_EDGE)],
            bsem)

    def bin_drain_write():
        pltpu.make_async_copy(
            bin_v,
            out.at[pl.ds(w0, C_BIN), pl.ds(2 * D_FEAT, D_EDGE)],
            bsem).wait()

    def slot(i, b, drain_w=True, idx_i=True, gather_i=True):
        sA = (b + 2) % RING
        if drain_w:
            drain_writes(sA)
        if idx_i:
            start_idx(i + 3, (b + 3) % RING)
        if gather_i:
            wait_idx(sA)
            start_gathers(i + 2, sA)
        drain_gathers(b)
        start_writes(i, b)

    start_idx(0, 0)
    start_idx(1, 1)
    start_idx(2, 2)
    wait_idx(0)
    start_gathers(0, 0)
    wait_idx(1)
    start_gathers(1, 1)

    bin_load(0)
    slot(0, 0, drain_w=False)
    slot(1, 1, drain_w=False)
    bin_drain_load()
    bin_write(0)
    slot(2, 2, drain_w=False)
    bin_drain_write()
    bin_load(1)
    slot(3, 3)
    bin_drain_load()
    bin_write(1)
    slot(4, 4)

    def round_body(r, carry):
        i0 = r * RING
        bin_drain_write()
        bin_load(2 * r)
        slot(i0 + 0, 0)
        slot(i0 + 1, 1)
        bin_drain_load()
        bin_write(2 * r)
        slot(i0 + 2, 2)
        bin_drain_write()
        bin_load(2 * r + 1)
        slot(i0 + 3, 3)
        bin_drain_load()
        bin_write(2 * r + 1)
        slot(i0 + 4, 4)
        return carry

    lax.fori_loop(1, N_ROUNDS - 1, round_body, 0)

    i0 = (N_ROUNDS - 1) * RING
    r = N_ROUNDS - 1
    bin_drain_write()
    bin_load(2 * r)
    slot(i0 + 0, 0)
    slot(i0 + 1, 1)
    bin_drain_load()
    bin_write(2 * r)
    slot(i0 + 2, 2, idx_i=False)
    bin_drain_write()
    bin_load(2 * r + 1)
    slot(i0 + 3, 3, idx_i=False, gather_i=False)
    bin_drain_load()
    bin_write(2 * r + 1)
    slot(i0 + 4, 4, idx_i=False, gather_i=False)

    drain_writes(2)
    drain_writes(3)
    drain_writes(4)
    bin_drain_write()


def kernel(unary, binary, index1, index2):
    return _join_sc(unary, binary, index1, index2)


# speedup vs baseline: 1.0020x; 1.0020x over previous
import functools

import jax
import jax.numpy as jnp
from jax import lax
from jax.experimental import pallas as pl
from jax.experimental.pallas import tpu as pltpu
from jax.experimental.pallas import tpu_sc as plsc

N_NODES = 10000
N_EDGES = 320000
D_FEAT = 128
D_EDGE = 16
D_OUT = 2 * D_FEAT + D_EDGE

NUM_CORES = 2
NUM_SUBCORES = 16
NW = NUM_CORES * NUM_SUBCORES
B_PER_W = N_EDGES // NW
CHUNK = 80
N_CHUNKS = B_PER_W // CHUNK
RING = 4
N_FULL_ROUNDS = 31

_mesh = plsc.VectorSubcoreMesh(core_axis_name="c", subcore_axis_name="s")


@functools.partial(
    pl.kernel,
    mesh=_mesh,
    out_type=jax.ShapeDtypeStruct((N_EDGES, D_OUT), jnp.float32),
    scratch_types=(
        [pltpu.VMEM((CHUNK,), jnp.int32) for _ in range(2 * RING)]
        + [
            pltpu.VMEM((2 * CHUNK,), jnp.int32),
            pltpu.VMEM((RING, 2 * CHUNK, D_FEAT), jnp.float32),
            pltpu.VMEM((CHUNK, D_EDGE), jnp.float32),
            pltpu.VMEM((CHUNK, D_EDGE), jnp.float32),
            pltpu.SemaphoreType.DMA((RING,)),
            pltpu.SemaphoreType.DMA((RING,)),
            pltpu.SemaphoreType.DMA((RING,)),
            pltpu.SemaphoreType.DMA((2,)),
        ]
    ),
)
def _join_sc(unary, binary, index1, index2, out, *refs):
    i1s = refs[0:RING]
    i2s = refs[RING:2 * RING]
    dummy_i, g_v, bin0, bin1, isem, gsem, wsem, bsem = refs[2 * RING:]
    bins = (bin0, bin1)

    wid = lax.axis_index("s") * NUM_CORES + lax.axis_index("c")
    w0 = wid * B_PER_W

    def start_idx(i, s):
        base = w0 + i * CHUNK
        pltpu.async_copy(index1.at[pl.ds(base, CHUNK)], i1s[s], isem.at[s])
        pltpu.async_copy(index2.at[pl.ds(base, CHUNK)], i2s[s], isem.at[s])

    def wait_idx(s):
        pltpu.make_async_copy(index1.at[pl.ds(w0, 2 * CHUNK)], dummy_i,
                              isem.at[s]).wait()

    def start_gathers(i, b):
        pltpu.async_copy(unary.at[i1s[b]],
                         g_v.at[b, pl.ds(0, CHUNK)], gsem.at[b])
        pltpu.async_copy(unary.at[i2s[b]],
                         g_v.at[b, pl.ds(CHUNK, CHUNK)], gsem.at[b])

    def drain_gathers(b):
        pltpu.make_async_copy(unary.at[i1s[b]],
                              g_v.at[b, pl.ds(0, CHUNK)], gsem.at[b]).wait()
        pltpu.make_async_copy(unary.at[i2s[b]],
                              g_v.at[b, pl.ds(CHUNK, CHUNK)], gsem.at[b]).wait()

    def start_writes(i, b):
        base = w0 + i * CHUNK
        pltpu.async_copy(g_v.at[b, pl.ds(0, CHUNK)],
                         out.at[pl.ds(base, CHUNK), pl.ds(0, D_FEAT)],
                         wsem.at[b])
        pltpu.async_copy(g_v.at[b, pl.ds(CHUNK, CHUNK)],
                         out.at[pl.ds(base, CHUNK), pl.ds(D_FEAT, D_FEAT)],
                         wsem.at[b])

    def drain_writes(b):
        pltpu.make_async_copy(g_v.at[b],
                              out.at[pl.ds(w0, 2 * CHUNK), pl.ds(0, D_FEAT)],
                              wsem.at[b]).wait()

    def bin_load(k, s):
        pltpu.async_copy(binary.at[pl.ds(w0 + k * CHUNK, CHUNK)],
                         bins[s], bsem.at[s])

    def bin_drain_load(s):
        pltpu.make_async_copy(binary.at[pl.ds(w0, CHUNK)],
                              bins[s], bsem.at[s]).wait()

    def bin_write(k, s):
        pltpu.async_copy(
            bins[s],
            out.at[pl.ds(w0 + k * CHUNK, CHUNK), pl.ds(2 * D_FEAT, D_EDGE)],
            bsem.at[s])

    def bin_drain_write(s):
        pltpu.make_async_copy(
            bins[s],
            out.at[pl.ds(w0, CHUNK), pl.ds(2 * D_FEAT, D_EDGE)],
            bsem.at[s]).wait()

    def slot(i, b, drain_w=True, idx_i=True, gather_i=True,
             bin_dw=True, bin_ld=True):
        sb = b % 2
        sA = (b + 2) % RING
        if drain_w:
            drain_writes(sA)
        if idx_i:
            start_idx(i + 3, (b + 3) % RING)
        if bin_dw:
            bin_drain_write(1 - sb)
        if bin_ld:
            bin_load(i + 1, 1 - sb)
        if gather_i:
            wait_idx(sA)
            start_gathers(i + 2, sA)
        drain_gathers(b)
        start_writes(i, b)
        bin_drain_load(sb)
        bin_write(i, sb)

    start_idx(0, 0)
    start_idx(1, 1)
    start_idx(2, 2)
    wait_idx(0)
    start_gathers(0, 0)
    wait_idx(1)
    start_gathers(1, 1)
    bin_load(0, 0)

    slot(0, 0, drain_w=False, bin_dw=False)
    slot(1, 1, drain_w=False)
    slot(2, 2)
    slot(3, 3)

    def round_body(r, carry):
        i0 = r * RING
        for b in range(RING):
            slot(i0 + b, b)
        return carry

    lax.fori_loop(1, N_FULL_ROUNDS - 1, round_body, 0)

    i0 = (N_FULL_ROUNDS - 1) * RING
    slot(i0 + 0, 0)
    slot(i0 + 1, 1)
    slot(i0 + 2, 2, idx_i=False)
    slot(i0 + 3, 3, idx_i=False, gather_i=False)
    slot(124, 0, idx_i=False, gather_i=False, bin_ld=False)

    drain_writes(3)
    drain_writes(0)
    bin_drain_write(0)


def kernel(unary, binary, index1, index2):
    return _join_sc(unary, binary, index1, index2)
